# baseline (speedup 1.0000x reference)
import functools

import jax
from jax import lax
import jax.numpy as jnp
from jax.experimental import pallas as pl
from jax.experimental.pallas import tpu as pltpu
from jax.experimental.pallas import tpu_sc as plsc

NC, NS = 2, 16
NW = NC * NS
CHUNK = 32
NBUF = 4


def kernel(input_ids, table):
    batch, seq = input_ids.shape
    n = batch * seq
    _, d = table.shape
    b_per_w = n // NW
    nchunk = b_per_w // CHUNK
    idx = input_ids.astype(jnp.int32)
    w_per_row = seq // b_per_w

    mesh = plsc.VectorSubcoreMesh(core_axis_name="c", subcore_axis_name="s")

    @functools.partial(
        pl.kernel,
        out_type=jax.ShapeDtypeStruct((n, d), table.dtype),
        mesh=mesh,
        scratch_types=[
            pltpu.VMEM((b_per_w,), jnp.int32),
            pltpu.VMEM((2, 2 * CHUNK, d), jnp.float32),
        ] + [pltpu.SemaphoreType.DMA] * 6,
    )
    def gather_kernel(tab_hbm, idx_hbm, out_hbm, idx_v, rows_v, *sems):
        gsems = sems[:4]
        osems = sems[4:]
        wid = lax.axis_index("s") * NC + lax.axis_index("c")
        base = wid * b_per_w
        row = wid // w_per_row
        col = (wid % w_per_row) * b_per_w
        head = NBUF * CHUNK
        pltpu.sync_copy(idx_hbm.at[row, pl.ds(col, head)],
                        idx_v.at[pl.ds(0, head)])

        def gather_cp(g, q, h):
            return pltpu.make_async_copy(
                tab_hbm.at[idx_v.at[pl.ds(g * CHUNK, CHUNK)]],
                rows_v.at[q, pl.ds(h * CHUNK, CHUNK)], gsems[2 * q + h])

        def out_cp(g2, q):
            return pltpu.make_async_copy(
                rows_v.at[q], out_hbm.at[pl.ds(base + g2 * CHUNK, 2 * CHUNK)],
                osems[q])

        for q in range(2):
            for h in range(2):
                gather_cp(2 * q + h, q, h).start()

        pltpu.sync_copy(idx_hbm.at[row, pl.ds(col + head, b_per_w - head)],
                        idx_v.at[pl.ds(head, b_per_w - head)])

        @pl.loop(0, nchunk, step=4)
        def _(c):
            for q in range(2):
                g0 = c + 2 * q
                gather_cp(g0, q, 0).wait()
                gather_cp(g0 + 1, q, 1).wait()
                out_cp(g0, q).start()

                @pl.when(g0 + 4 < nchunk)
                def _():
                    out_cp(g0, q).wait()
                    gather_cp(g0 + 4, q, 0).start()
                    gather_cp(g0 + 5, q, 1).start()

        for q in range(2):
            out_cp(nchunk - 4 + 2 * q, q).wait()

    out = gather_kernel(table, idx)
    return out.reshape(batch, seq, d)
